# baseline (speedup 1.0000x reference)
import functools

import jax
import jax.numpy as jnp
from jax import lax
from jax.experimental import pallas as pl
from jax.experimental.pallas import tpu as pltpu
from jax.experimental.pallas import tpu_sc as plsc

_POOL = 10000
_L = 5
_C = 768
_B = 256
_T = 197
_SLOTS = 2 * _L
_OUT_T = _SLOTS + _T

_BT = 16
_PC = 2000
_NP = _POOL // _PC

_NC, _NS = 2, 16
_NW = _NC * _NS
_BPW = _B // _NW


def _mean_copy_body(x_ref, out_ref, xn_ref):
    x = x_ref[...]
    out_ref[:, _SLOTS:, :] = x
    m = jnp.sum(x, axis=1) * (1.0 / _T)
    ssq = jnp.sum(m * m, axis=1, keepdims=True)
    xn_ref[...] = m * lax.rsqrt(jnp.maximum(ssq, 1e-12))


def _mean_copy(x_embed):
    return pl.pallas_call(
        _mean_copy_body,
        grid=(_B // _BT,),
        in_specs=[pl.BlockSpec((_BT, _T, _C), lambda b: (b, 0, 0))],
        out_specs=[
            pl.BlockSpec((_BT, _OUT_T, _C), lambda b: (b, 0, 0)),
            pl.BlockSpec((_BT, _C), lambda b: (b, 0)),
        ],
        out_shape=[
            jax.ShapeDtypeStruct((_B, _OUT_T, _C), jnp.float32),
            jax.ShapeDtypeStruct((_B, _C), jnp.float32),
        ],
    )(x_embed)


def _sim_body(xn_ref, pk_ref, idx_ref, row_ref, smax_ref, m_scr, a_scr, r_scr):
    p = pl.program_id(0)
    xn = xn_ref[...]
    pk = pk_ref[...]
    inv = lax.rsqrt(jnp.maximum(jnp.sum(pk * pk, axis=1, keepdims=True), 1e-12))
    pkn = pk * inv
    sim = lax.dot_general(xn, pkn, (((1,), (1,)), ((), ())),
                          preferred_element_type=jnp.float32)
    lmax = jnp.max(sim, axis=1)[:, None]
    larg = jnp.argmax(sim, axis=1).astype(jnp.int32)[:, None]
    onehot = (lax.broadcasted_iota(jnp.int32, sim.shape, 1)
              == larg).astype(jnp.float32)
    lrow = lax.dot_general(onehot, pk, (((1,), (0,)), ((), ())),
                           preferred_element_type=jnp.float32)

    @pl.when(p == 0)
    def _():
        m_scr[...] = lmax
        a_scr[...] = larg
        r_scr[...] = lrow

    @pl.when(p > 0)
    def _():
        better = lmax > m_scr[...]
        m_scr[...] = jnp.where(better, lmax, m_scr[...])
        a_scr[...] = jnp.where(better, larg + p * _PC, a_scr[...])
        r_scr[...] = jnp.where(better, lrow, r_scr[...])

    @pl.when(p == _NP - 1)
    def _():
        idx_ref[...] = a_scr[...]
        row_ref[...] = r_scr[...]
        smax_ref[...] = m_scr[...]


def _sim(xn, prompt_key):
    return pl.pallas_call(
        _sim_body,
        grid=(_NP,),
        in_specs=[
            pl.BlockSpec((_B, _C), lambda p: (0, 0)),
            pl.BlockSpec((_PC, _C), lambda p: (p, 0)),
        ],
        out_specs=[
            pl.BlockSpec((_B, 1), lambda p: (0, 0)),
            pl.BlockSpec((_B, _C), lambda p: (0, 0)),
            pl.BlockSpec((_B, 1), lambda p: (0, 0)),
        ],
        out_shape=[
            jax.ShapeDtypeStruct((_B, 1), jnp.int32),
            jax.ShapeDtypeStruct((_B, _C), jnp.float32),
            jax.ShapeDtypeStruct((_B, 1), jnp.float32),
        ],
        scratch_shapes=[
            pltpu.VMEM((_B, 1), jnp.float32),
            pltpu.VMEM((_B, 1), jnp.int32),
            pltpu.VMEM((_B, _C), jnp.float32),
        ],
    )(xn, prompt_key)


def _res_body(xn_ref, row_ref, smax_ref, rpk_ref, ridx_ref, tot_ref,
              m_scr, a_scr):
    p = pl.program_id(0)
    res = row_ref[...] - xn_ref[...]
    rpk = rpk_ref[...]
    inv = lax.rsqrt(jnp.maximum(jnp.sum(rpk * rpk, axis=1, keepdims=True), 1e-12))
    rpkn = rpk * inv
    sim = lax.dot_general(res, rpkn, (((1,), (1,)), ((), ())),
                          preferred_element_type=jnp.float32)
    lmax = jnp.max(sim, axis=1)[:, None]
    larg = jnp.argmax(sim, axis=1).astype(jnp.int32)[:, None]

    @pl.when(p == 0)
    def _():
        m_scr[...] = lmax
        a_scr[...] = larg

    @pl.when(p > 0)
    def _():
        better = lmax > m_scr[...]
        m_scr[...] = jnp.where(better, lmax, m_scr[...])
        a_scr[...] = jnp.where(better, larg + p * _PC, a_scr[...])

    @pl.when(p == _NP - 1)
    def _():
        ridx_ref[...] = a_scr[...]
        tot_ref[...] = ((jnp.sum(m_scr[...]) + jnp.sum(smax_ref[...]))
                        * (1.0 / _B))[None, None]


def _res(xn, row, smax, residual_prompt_key):
    return pl.pallas_call(
        _res_body,
        grid=(_NP,),
        in_specs=[
            pl.BlockSpec((_B, _C), lambda p: (0, 0)),
            pl.BlockSpec((_B, _C), lambda p: (0, 0)),
            pl.BlockSpec((_B, 1), lambda p: (0, 0)),
            pl.BlockSpec((_PC, _C), lambda p: (p, 0)),
        ],
        out_specs=[
            pl.BlockSpec((_B, 1), lambda p: (0, 0)),
            pl.BlockSpec((1, 1), lambda p: (0, 0)),
        ],
        out_shape=[
            jax.ShapeDtypeStruct((_B, 1), jnp.int32),
            jax.ShapeDtypeStruct((1, 1), jnp.float32),
        ],
        scratch_shapes=[
            pltpu.VMEM((_B, 1), jnp.float32),
            pltpu.VMEM((_B, 1), jnp.int32),
        ],
    )(xn, row, smax, residual_prompt_key)


def _gather_sc_body(prompt_hbm, rprompt_hbm, idx_hbm, ridx_hbm, out_hbm,
                    idx_v, ridx_v, rows_v, rrows_v, sem1, sem2):
    wid = lax.axis_index("s") * _NC + lax.axis_index("c")
    base = wid * _BPW
    pltpu.sync_copy(idx_hbm.at[pl.ds(base, _BPW)], idx_v)
    pltpu.sync_copy(ridx_hbm.at[pl.ds(base, _BPW)], ridx_v)
    cp1 = pltpu.async_copy(prompt_hbm.at[idx_v], rows_v, sem1)
    cp2 = pltpu.async_copy(rprompt_hbm.at[ridx_v], rrows_v, sem2)
    cp1.wait()
    pltpu.sync_copy(rows_v, out_hbm.at[pl.ds(base, _BPW), pl.ds(_L * _C, _L * _C)])
    cp2.wait()
    pltpu.sync_copy(rrows_v, out_hbm.at[pl.ds(base, _BPW), pl.ds(0, _L * _C)])


@functools.cache
def _gather_sc_fn():
    return pl.kernel(
        _gather_sc_body,
        out_type=jax.ShapeDtypeStruct((_B, _SLOTS * _C), jnp.float32),
        mesh=plsc.VectorSubcoreMesh(core_axis_name="c", subcore_axis_name="s",
                                    num_cores=_NC, num_subcores=_NS),
        scratch_types=[
            pltpu.VMEM((_BPW,), jnp.int32),
            pltpu.VMEM((_BPW,), jnp.int32),
            pltpu.VMEM((_BPW, _L * _C), jnp.float32),
            pltpu.VMEM((_BPW, _L * _C), jnp.float32),
            pltpu.SemaphoreType.DMA,
            pltpu.SemaphoreType.DMA,
        ],
    )


def _insert_body(o_any, g_ref, o_ref):
    del o_any
    o_ref[...] = g_ref[...]


def _insert(out0, gath):
    flat = pl.pallas_call(
        _insert_body,
        grid=(4,),
        in_specs=[
            pl.BlockSpec(memory_space=pl.ANY),
            pl.BlockSpec((_B // 4, _SLOTS * _C), lambda b: (b, 0)),
        ],
        out_specs=pl.BlockSpec((_B // 4, _SLOTS * _C), lambda b: (b, 0)),
        out_shape=jax.ShapeDtypeStruct((_B, _OUT_T * _C), jnp.float32),
        input_output_aliases={0: 0},
    )(out0.reshape(_B, _OUT_T * _C), gath.reshape(_B, _SLOTS * _C))
    return flat.reshape(_B, _OUT_T, _C)


def kernel(x_embed, prompt, prompt_key, residual_prompt, residual_prompt_key,
           iseval, task_id):
    del iseval, task_id
    out0, xn = _mean_copy(x_embed)
    idx, row, smax = _sim(xn, prompt_key)
    ridx, tot = _res(xn, row, smax, residual_prompt_key)
    gath = _gather_sc_fn()(prompt.reshape(_POOL, _L * _C),
                           residual_prompt.reshape(_POOL, _L * _C),
                           idx.reshape(_B), ridx.reshape(_B))
    out = _insert(out0, gath)
    return out, tot.reshape(())


# speedup vs baseline: 1.1418x; 1.1418x over previous
import functools

import jax
import jax.numpy as jnp
from jax import lax
from jax.experimental import pallas as pl
from jax.experimental.pallas import tpu as pltpu
from jax.experimental.pallas import tpu_sc as plsc

_POOL = 10000
_L = 5
_C = 768
_B = 256
_T = 197
_SLOTS = 2 * _L
_OUT_T = _SLOTS + _T

_BT = 16
_PC = 2000
_NP = _POOL // _PC

_NC, _NS = 2, 16
_NW = _NC * _NS
_BPW = _B // _NW


def _copy_body(x_ref, out_ref):
    out_ref[:, _SLOTS:, :] = x_ref[...]


def _copy(x_embed):
    return pl.pallas_call(
        _copy_body,
        grid=(_B // _BT,),
        in_specs=[pl.BlockSpec((_BT, _T, _C), lambda b: (b, 0, 0))],
        out_specs=pl.BlockSpec((_BT, _OUT_T, _C), lambda b: (b, 0, 0)),
        out_shape=jax.ShapeDtypeStruct((_B, _OUT_T, _C), jnp.float32),
    )(x_embed)


def _sim_body(xn_ref, pk_ref, inv_ref, idx_ref, row_ref, smax_ref,
              m_scr, a_scr, r_scr):
    p = pl.program_id(0)
    xn = xn_ref[...]
    pk = pk_ref[...]
    pkn = pk * inv_ref[...]
    sim = lax.dot_general(xn, pkn, (((1,), (1,)), ((), ())),
                          preferred_element_type=jnp.float32)
    lmax = jnp.max(sim, axis=1)[:, None]
    larg = jnp.argmax(sim, axis=1).astype(jnp.int32)[:, None]
    onehot = (lax.broadcasted_iota(jnp.int32, sim.shape, 1)
              == larg).astype(jnp.float32)
    lrow = lax.dot_general(onehot, pk, (((1,), (0,)), ((), ())),
                           precision=lax.Precision.HIGHEST,
                           preferred_element_type=jnp.float32)

    @pl.when(p == 0)
    def _():
        m_scr[...] = lmax
        a_scr[...] = larg
        r_scr[...] = lrow

    @pl.when(p > 0)
    def _():
        better = lmax > m_scr[...]
        m_scr[...] = jnp.where(better, lmax, m_scr[...])
        a_scr[...] = jnp.where(better, larg + p * _PC, a_scr[...])
        r_scr[...] = jnp.where(better, lrow, r_scr[...])

    @pl.when(p == _NP - 1)
    def _():
        idx_ref[...] = a_scr[...]
        row_ref[...] = r_scr[...]
        smax_ref[...] = m_scr[...]


def _sim(xn, prompt_key, inv_pk):
    return pl.pallas_call(
        _sim_body,
        grid=(_NP,),
        in_specs=[
            pl.BlockSpec((_B, _C), lambda p: (0, 0)),
            pl.BlockSpec((_PC, _C), lambda p: (p, 0)),
            pl.BlockSpec((_PC, 1), lambda p: (p, 0)),
        ],
        out_specs=[
            pl.BlockSpec((_B, 1), lambda p: (0, 0)),
            pl.BlockSpec((_B, _C), lambda p: (0, 0)),
            pl.BlockSpec((_B, 1), lambda p: (0, 0)),
        ],
        out_shape=[
            jax.ShapeDtypeStruct((_B, 1), jnp.int32),
            jax.ShapeDtypeStruct((_B, _C), jnp.float32),
            jax.ShapeDtypeStruct((_B, 1), jnp.float32),
        ],
        scratch_shapes=[
            pltpu.VMEM((_B, 1), jnp.float32),
            pltpu.VMEM((_B, 1), jnp.int32),
            pltpu.VMEM((_B, _C), jnp.float32),
        ],
    )(xn, prompt_key, inv_pk)


def _res_body(xn_ref, row_ref, smax_ref, rpk_ref, inv_ref, ridx_ref, tot_ref,
              m_scr, a_scr):
    p = pl.program_id(0)
    res = row_ref[...] - xn_ref[...]
    rpk = rpk_ref[...]
    rpkn = rpk * inv_ref[...]
    sim = lax.dot_general(res, rpkn, (((1,), (1,)), ((), ())),
                          preferred_element_type=jnp.float32)
    lmax = jnp.max(sim, axis=1)[:, None]
    larg = jnp.argmax(sim, axis=1).astype(jnp.int32)[:, None]

    @pl.when(p == 0)
    def _():
        m_scr[...] = lmax
        a_scr[...] = larg

    @pl.when(p > 0)
    def _():
        better = lmax > m_scr[...]
        m_scr[...] = jnp.where(better, lmax, m_scr[...])
        a_scr[...] = jnp.where(better, larg + p * _PC, a_scr[...])

    @pl.when(p == _NP - 1)
    def _():
        ridx_ref[...] = a_scr[...]
        tot_ref[...] = ((jnp.sum(m_scr[...]) + jnp.sum(smax_ref[...]))
                        * (1.0 / _B))[None, None]


def _res(xn, row, smax, residual_prompt_key, inv_rpk):
    return pl.pallas_call(
        _res_body,
        grid=(_NP,),
        in_specs=[
            pl.BlockSpec((_B, _C), lambda p: (0, 0)),
            pl.BlockSpec((_B, _C), lambda p: (0, 0)),
            pl.BlockSpec((_B, 1), lambda p: (0, 0)),
            pl.BlockSpec((_PC, _C), lambda p: (p, 0)),
            pl.BlockSpec((_PC, 1), lambda p: (p, 0)),
        ],
        out_specs=[
            pl.BlockSpec((_B, 1), lambda p: (0, 0)),
            pl.BlockSpec((1, 1), lambda p: (0, 0)),
        ],
        out_shape=[
            jax.ShapeDtypeStruct((_B, 1), jnp.int32),
            jax.ShapeDtypeStruct((1, 1), jnp.float32),
        ],
        scratch_shapes=[
            pltpu.VMEM((_B, 1), jnp.float32),
            pltpu.VMEM((_B, 1), jnp.int32),
        ],
    )(xn, row, smax, residual_prompt_key, inv_rpk)


_BPC = _B // _NC


def _gather_sc_body(prompt_hbm, rprompt_hbm, idx_hbm, ridx_hbm,
                    pout_hbm, rout_hbm,
                    idx_s, ridx_s, sem1, sem2):
    c = lax.axis_index("c")
    base = c * _BPC
    pltpu.sync_copy(idx_hbm.at[pl.ds(base, _BPC)], idx_s)
    pltpu.sync_copy(ridx_hbm.at[pl.ds(base, _BPC)], ridx_s)

    def step(j, _):
        pltpu.async_copy(prompt_hbm.at[idx_s[j]],
                         pout_hbm.at[base + j], sem1)
        pltpu.async_copy(rprompt_hbm.at[ridx_s[j]],
                         rout_hbm.at[base + j], sem2)
        return ()

    lax.fori_loop(0, _BPC, step, ())
    pltpu.make_async_copy(prompt_hbm.at[pl.ds(0, _BPC)],
                          pout_hbm.at[pl.ds(base, _BPC)], sem1).wait()
    pltpu.make_async_copy(rprompt_hbm.at[pl.ds(0, _BPC)],
                          rout_hbm.at[pl.ds(base, _BPC)], sem2).wait()


@functools.cache
def _gather_sc_fn():
    return pl.kernel(
        _gather_sc_body,
        out_type=[
            jax.ShapeDtypeStruct((_B, _L, _C), jnp.float32),
            jax.ShapeDtypeStruct((_B, _L, _C), jnp.float32),
        ],
        mesh=plsc.ScalarSubcoreMesh(axis_name="c", num_cores=_NC),
        scratch_types=[
            pltpu.SMEM((_BPC,), jnp.int32),
            pltpu.SMEM((_BPC,), jnp.int32),
            pltpu.SemaphoreType.DMA,
            pltpu.SemaphoreType.DMA,
        ],
    )


_BT2 = 64


def _insert_body(o_any, r_ref, p_ref, x_ref, o_ref, scr, sem):
    del o_any
    i = pl.program_id(0)
    scr[:, 0:_L, :] = r_ref[...]
    scr[:, _L:_SLOTS, :] = p_ref[...]
    scr[:, _SLOTS:16, :] = x_ref[:, 0:16 - _SLOTS, :]
    pltpu.async_copy(
        scr, o_ref.at[pl.ds(i * _BT2, _BT2), pl.ds(0, 16)], sem).wait()


def _insert(out0, gath_r, gath_p, x_embed):
    return pl.pallas_call(
        _insert_body,
        grid=(_B // _BT2,),
        in_specs=[
            pl.BlockSpec(memory_space=pl.ANY),
            pl.BlockSpec((_BT2, _L, _C), lambda b: (b, 0, 0)),
            pl.BlockSpec((_BT2, _L, _C), lambda b: (b, 0, 0)),
            pl.BlockSpec((_BT2, 8, _C), lambda b: (b, 0, 0)),
        ],
        out_specs=pl.BlockSpec(memory_space=pl.ANY),
        out_shape=jax.ShapeDtypeStruct((_B, _OUT_T, _C), jnp.float32),
        scratch_shapes=[
            pltpu.VMEM((_BT2, 16, _C), jnp.float32),
            pltpu.SemaphoreType.DMA,
        ],
        input_output_aliases={0: 0},
    )(out0, gath_r, gath_p, x_embed)


def kernel(x_embed, prompt, prompt_key, residual_prompt, residual_prompt_key,
           iseval, task_id):
    del iseval, task_id
    xm = jnp.mean(x_embed, axis=1)
    xn = xm * lax.rsqrt(jnp.maximum(
        jnp.sum(xm * xm, axis=1, keepdims=True), 1e-12))
    inv_pk = lax.rsqrt(jnp.maximum(
        jnp.sum(prompt_key * prompt_key, axis=1, keepdims=True), 1e-12))
    inv_rpk = lax.rsqrt(jnp.maximum(
        jnp.sum(residual_prompt_key * residual_prompt_key, axis=1,
                keepdims=True), 1e-12))
    out0 = _copy(x_embed)
    idx, row, smax = _sim(xn, prompt_key, inv_pk)
    ridx, tot = _res(xn, row, smax, residual_prompt_key, inv_rpk)
    gath_p, gath_r = _gather_sc_fn()(prompt, residual_prompt,
                                     idx.reshape(_B), ridx.reshape(_B))
    out = _insert(out0, gath_r, gath_p, x_embed)
    return out, tot.reshape(())
